# baseline (speedup 1.0000x reference)
import functools

import jax
import jax.numpy as jnp
from jax import lax
from jax.experimental import pallas as pl
from jax.experimental.pallas import tpu as pltpu
from jax.experimental.pallas import tpu_sc as plsc

_R1 = (0, 0, 0, 1, 2, 2, 2)
_C4 = (0, 0, 0, 1, 2, 2, 2)

_NC, _NS, _L = 2, 16, 16
_NW = _NC * _NS

_OFF_V1 = 8
_OFF_B1 = 15
_OFF_V2 = 18
_OFF_B2 = 21
_OFF_V3 = 22
_OFF_B3 = 25
_OFF_V4 = 28
_OFF_B4 = 35
_PLEN = 48


def _splat(p_ref, off):
    return plsc.load_gather(p_ref, [jnp.full((_L,), off, jnp.int32)])


def _sae_body(nwords, x_hbm, p_hbm, o_hbm, x_v, o_v, p_v):
    wid = lax.axis_index("s") * _NC + lax.axis_index("c")
    base = wid * nwords
    pltpu.sync_copy(x_hbm.at[pl.ds(base, nwords)], x_v)
    pltpu.sync_copy(p_hbm, p_v)

    w = [_splat(p_v, _OFF_V1 + k) * _splat(p_v, _OFF_V2 + _R1[k])
         for k in range(7)]
    c = (_splat(p_v, _OFF_V2 + 0) * _splat(p_v, _OFF_B1 + 0)
         + _splat(p_v, _OFF_V2 + 1) * _splat(p_v, _OFF_B1 + 1)
         + _splat(p_v, _OFF_V2 + 2) * _splat(p_v, _OFF_B1 + 2)
         + _splat(p_v, _OFF_B2))
    a = [_splat(p_v, _OFF_V4 + j) * _splat(p_v, _OFF_V3 + _C4[j])
         for j in range(7)]
    e = [_splat(p_v, _OFF_V4 + j) * _splat(p_v, _OFF_B3 + _C4[j])
         + _splat(p_v, _OFF_B4 + j) for j in range(7)]

    iota7 = lax.iota(jnp.int32, _L) * 7
    ngroups = nwords // (7 * _L)

    def group(g, carry):
        idx0 = iota7 + g * (7 * _L)
        s = c
        for k in range(7):
            xk = plsc.load_gather(x_v, [idx0 + k])
            s = s + xk * w[k]
        for j in range(7):
            plsc.store_scatter(o_v, [idx0 + j], s * a[j] + e[j])
        return carry

    lax.fori_loop(0, ngroups, group, 0)
    pltpu.sync_copy(o_v, o_hbm.at[pl.ds(base, nwords)])


def kernel(x, v1, b1, v2, b2, v3, b3, v4, b4):
    B = x.shape[0]
    rows_align = _NW * _L
    Bp = ((B + rows_align - 1) // rows_align) * rows_align
    xf = x.reshape(-1)
    if Bp != B:
        xf = jnp.pad(xf, (0, (Bp - B) * 7))
    nwords = (Bp * 7) // _NW

    params = jnp.concatenate(
        [jnp.zeros((_OFF_V1,), jnp.float32),
         v1, b1, v2, b2, v3, b3, v4, b4,
         jnp.zeros((_PLEN - _OFF_V1 - 34,), jnp.float32)])

    mesh = plsc.VectorSubcoreMesh(
        core_axis_name="c", subcore_axis_name="s",
        num_cores=_NC, num_subcores=_NS)
    out = pl.kernel(
        functools.partial(_sae_body, nwords),
        out_type=jax.ShapeDtypeStruct((Bp * 7,), jnp.float32),
        mesh=mesh,
        scratch_types=[
            pltpu.VMEM((nwords,), jnp.float32),
            pltpu.VMEM((nwords,), jnp.float32),
            pltpu.VMEM((_PLEN,), jnp.float32),
        ],
        compiler_params=pltpu.CompilerParams(needs_layout_passes=False),
    )(xf, params)
    return out[: B * 7].reshape(B, 7)


# speedup vs baseline: 1.0013x; 1.0013x over previous
import functools

import jax
import jax.numpy as jnp
from jax import lax
from jax.experimental import pallas as pl
from jax.experimental.pallas import tpu as pltpu
from jax.experimental.pallas import tpu_sc as plsc

_R1 = (0, 0, 0, 1, 2, 2, 2)
_C4 = (0, 0, 0, 1, 2, 2, 2)

_NC, _NS, _L = 2, 16, 16
_NW = _NC * _NS

_OFF_V1 = 8
_OFF_B1 = 15
_OFF_V2 = 18
_OFF_B2 = 21
_OFF_V3 = 22
_OFF_B3 = 25
_OFF_V4 = 28
_OFF_B4 = 35
_PLEN = 48


def _splat(p_ref, off):
    return plsc.load_gather(p_ref, [jnp.full((_L,), off, jnp.int32)])


def _sae_body(nwords, x_hbm, p_hbm, o_hbm, x_v, o_v, p_v):
    wid = lax.axis_index("s") * _NC + lax.axis_index("c")
    base = wid * nwords
    pltpu.sync_copy(x_hbm.at[pl.ds(base, nwords)], x_v)
    pltpu.sync_copy(p_hbm, p_v)

    w = [_splat(p_v, _OFF_V1 + k) * _splat(p_v, _OFF_V2 + _R1[k])
         for k in range(7)]
    c = (_splat(p_v, _OFF_V2 + 0) * _splat(p_v, _OFF_B1 + 0)
         + _splat(p_v, _OFF_V2 + 1) * _splat(p_v, _OFF_B1 + 1)
         + _splat(p_v, _OFF_V2 + 2) * _splat(p_v, _OFF_B1 + 2)
         + _splat(p_v, _OFF_B2))
    a = [_splat(p_v, _OFF_V4 + j) * _splat(p_v, _OFF_V3 + _C4[j])
         for j in range(7)]
    e = [_splat(p_v, _OFF_V4 + j) * _splat(p_v, _OFF_B3 + _C4[j])
         + _splat(p_v, _OFF_B4 + j) for j in range(7)]

    iota7 = lax.iota(jnp.int32, _L) * 7
    ngroups = nwords // (7 * _L)

    @plsc.parallel_loop(0, ngroups * 7 * _L, 7 * _L, unroll=4)
    def group(gbase):
        idx0 = iota7 + gbase
        s = c
        for k in range(7):
            xk = plsc.load_gather(x_v, [idx0 + k])
            s = s + xk * w[k]
        for j in range(7):
            plsc.store_scatter(o_v, [idx0 + j], s * a[j] + e[j])
    pltpu.sync_copy(o_v, o_hbm.at[pl.ds(base, nwords)])


def kernel(x, v1, b1, v2, b2, v3, b3, v4, b4):
    B = x.shape[0]
    rows_align = _NW * _L
    Bp = ((B + rows_align - 1) // rows_align) * rows_align
    xf = x.reshape(-1)
    if Bp != B:
        xf = jnp.pad(xf, (0, (Bp - B) * 7))
    nwords = (Bp * 7) // _NW

    params = jnp.concatenate(
        [jnp.zeros((_OFF_V1,), jnp.float32),
         v1, b1, v2, b2, v3, b3, v4, b4,
         jnp.zeros((_PLEN - _OFF_V1 - 34,), jnp.float32)])

    mesh = plsc.VectorSubcoreMesh(
        core_axis_name="c", subcore_axis_name="s",
        num_cores=_NC, num_subcores=_NS)
    out = pl.kernel(
        functools.partial(_sae_body, nwords),
        out_type=jax.ShapeDtypeStruct((Bp * 7,), jnp.float32),
        mesh=mesh,
        scratch_types=[
            pltpu.VMEM((nwords,), jnp.float32),
            pltpu.VMEM((nwords,), jnp.float32),
            pltpu.VMEM((_PLEN,), jnp.float32),
        ],
        compiler_params=pltpu.CompilerParams(needs_layout_passes=False),
    )(xf, params)
    return out[: B * 7].reshape(B, 7)


# speedup vs baseline: 1.4245x; 1.4227x over previous
import functools

import jax
import jax.numpy as jnp
from jax import lax
from jax.experimental import pallas as pl
from jax.experimental.pallas import tpu as pltpu
from jax.experimental.pallas import tpu_sc as plsc

_R1 = (0, 0, 0, 1, 2, 2, 2)
_C4 = (0, 0, 0, 1, 2, 2, 2)

_NC, _NS, _L = 2, 16, 16
_NW = _NC * _NS
_CHROWS = 128

_OFF_V1 = 8
_OFF_B1 = 15
_OFF_V2 = 18
_OFF_B2 = 21
_OFF_V3 = 22
_OFF_B3 = 25
_OFF_V4 = 28
_OFF_B4 = 35
_PLEN = 48


def _splat(p_ref, off):
    return plsc.load_gather(p_ref, [jnp.full((_L,), off, jnp.int32)])


def _sae_body(nrows, x_hbm, p_hbm, o_hbm, xv, ov, p_v,
              sin0, sin1, sout0, sout1):
    wid = lax.axis_index("s") * _NC + lax.axis_index("c")
    base = wid * nrows
    nch = nrows // _CHROWS
    sin = (sin0, sin1)
    sout = (sout0, sout1)

    pltpu.sync_copy(p_hbm, p_v)
    w = [_splat(p_v, _OFF_V1 + k) * _splat(p_v, _OFF_V2 + _R1[k])
         for k in range(7)]
    c = (_splat(p_v, _OFF_V2 + 0) * _splat(p_v, _OFF_B1 + 0)
         + _splat(p_v, _OFF_V2 + 1) * _splat(p_v, _OFF_B1 + 1)
         + _splat(p_v, _OFF_V2 + 2) * _splat(p_v, _OFF_B1 + 2)
         + _splat(p_v, _OFF_B2))
    a = [_splat(p_v, _OFF_V4 + j) * _splat(p_v, _OFF_V3 + _C4[j])
         for j in range(7)]
    e = [_splat(p_v, _OFF_V4 + j) * _splat(p_v, _OFF_B3 + _C4[j])
         + _splat(p_v, _OFF_B4 + j) for j in range(7)]

    iota16 = lax.iota(jnp.int32, _L)
    kidx = [jnp.full((_L,), k, jnp.int32) for k in range(7)]

    def in_copy(i, b):
        return pltpu.async_copy(
            x_hbm.at[pl.ds(base + i * _CHROWS, _CHROWS)], xv[b], sin[b])

    def out_copy(i, b):
        return pltpu.async_copy(
            ov[b], o_hbm.at[pl.ds(base + i * _CHROWS, _CHROWS)], sout[b])

    cp_in = {0: in_copy(0, 0)}
    cp_out = {}
    for i in range(nch):
        b = i % 2
        if i + 1 < nch:
            cp_in[(i + 1) % 2] = in_copy(i + 1, (i + 1) % 2)
        cp_in[b].wait()
        if i >= 2:
            cp_out[b].wait()
        for t in range(_CHROWS // _L):
            ridx = iota16 + (t * _L)
            s = c
            for k in range(7):
                s = s + plsc.load_gather(xv[b], [ridx, kidx[k]]) * w[k]
            for j in range(7):
                plsc.store_scatter(ov[b], [ridx, kidx[j]], s * a[j] + e[j])
        cp_out[b] = out_copy(i, b)
    cp_out[(nch - 2) % 2].wait()
    cp_out[(nch - 1) % 2].wait()


def kernel(x, v1, b1, v2, b2, v3, b3, v4, b4):
    B = x.shape[0]
    nrows = B // _NW
    assert B % (_NW * _CHROWS) == 0, "batch must split evenly across subcores"

    params = jnp.concatenate(
        [jnp.zeros((_OFF_V1,), jnp.float32),
         v1, b1, v2, b2, v3, b3, v4, b4,
         jnp.zeros((_PLEN - _OFF_V1 - 34,), jnp.float32)])

    mesh = plsc.VectorSubcoreMesh(
        core_axis_name="c", subcore_axis_name="s",
        num_cores=_NC, num_subcores=_NS)
    out = pl.kernel(
        functools.partial(_sae_body, nrows),
        out_type=jax.ShapeDtypeStruct((B, 7), jnp.float32),
        mesh=mesh,
        scratch_types=[
            [pltpu.VMEM((_CHROWS, 7), jnp.float32) for _ in range(2)],
            [pltpu.VMEM((_CHROWS, 7), jnp.float32) for _ in range(2)],
            pltpu.VMEM((_PLEN,), jnp.float32),
            pltpu.SemaphoreType.DMA,
            pltpu.SemaphoreType.DMA,
            pltpu.SemaphoreType.DMA,
            pltpu.SemaphoreType.DMA,
        ],
        compiler_params=pltpu.CompilerParams(
            needs_layout_passes=False, use_tc_tiling_on_sc=True),
    )(x, params)
    return out
